# Initial kernel scaffold; baseline (speedup 1.0000x reference)
import jax
import jax.numpy as jnp
from jax.experimental import pallas as pl


def kernel(f_a, position_embedding, iou, WG_w, WG_b, WK_w, WK_b, WQ_w, WQ_b, conv_w, conv_b):
    raise NotImplementedError("write your pallas kernel here")


# speedup vs baseline: 6.9061x; 6.9061x over previous
import math

import jax
import jax.numpy as jnp
from jax.experimental import pallas as pl
from jax.experimental.pallas import tpu as pltpu

N = 512
C = 2
F = 1024
GEO = 64
G = 16
DG = 64
TOPK = 10
LOG1EM6 = float(math.log(1e-6))


def _proj_kernel(x_ref, xt_ref, wqp_ref, bqp_ref, wk_ref, bk_ref,
                 yqp_ref, kt_ref):
    yqp_ref[...] = (
        jnp.dot(x_ref[...], wqp_ref[...], preferred_element_type=jnp.float32)
        + bqp_ref[...]
    )
    kt_ref[...] = (
        jnp.dot(wk_ref[...], xt_ref[...], preferred_element_type=jnp.float32)
        + bk_ref[...]
    )


def _attn_kernel(pe_ref, iou_ref, yq_ref, yp_ref, kt_ref, wgw_ref, wgb_ref,
                 cb_ref, o_ref, pv_s):
    bn = iou_ref.shape[1]

    @pl.when(pl.program_id(1) == 0)
    def _load_pv():
        pv_s[...] = jnp.transpose(yp_ref[...].reshape(N, G, DG), (1, 0, 2))

    wg = jnp.dot(pe_ref[0], wgw_ref[...], preferred_element_type=jnp.float32)
    gate = jnp.maximum(wg + wgb_ref[...], 1e-6)
    gate_t = jnp.transpose(gate.reshape(bn, N, G), (2, 0, 1))

    wq_t = jnp.transpose(yq_ref[...].reshape(bn, G, DG), (1, 0, 2))
    wk_t = kt_ref[...].reshape(G, DG, N)
    aff = jax.lax.dot_general(
        wq_t, wk_t, (((2,), (1,)), ((0,), (0,))),
        preferred_element_type=jnp.float32) * 0.125

    al = aff + jnp.where(iou_ref[0] >= 1e-6, 0.0, LOG1EM6)[None]
    amax = jnp.max(al, axis=-1, keepdims=True)
    p = gate_t * jnp.exp(al - amax)

    cur = p
    for _ in range(TOPK - 1):
        mx = jnp.max(cur, axis=-1, keepdims=True)
        cur = jnp.where(cur == mx, -1.0, cur)
    thr = jnp.max(cur, axis=-1, keepdims=True)

    w = jnp.where(p >= thr, p, 0.0)
    z = jnp.sum(w, axis=-1, keepdims=True)

    out = jax.lax.dot_general(
        w, pv_s[...], (((2,), (1,)), ((0,), (0,))),
        preferred_element_type=jnp.float32) / z

    out_t = jnp.transpose(out, (1, 0, 2)).reshape(bn, G * DG)
    o_ref[...] = out_t + cb_ref[...]


@jax.jit
def kernel(f_a, position_embedding, iou, WG_w, WG_b, WK_w, WK_b, WQ_w, WQ_b,
           conv_w, conv_b):
    f32 = jnp.float32
    x = jnp.transpose(f_a, (1, 0, 2)).reshape(C * N, F)
    wp = conv_w[:, :, 0, 0]
    wqp = jnp.concatenate([WQ_w.T, wp.T], axis=1)
    bqp = jnp.concatenate([WQ_b, jnp.zeros((F,), f32)])[None, :]

    yqp, kt = pl.pallas_call(
        _proj_kernel,
        out_shape=(
            jax.ShapeDtypeStruct((C * N, 2 * F), f32),
            jax.ShapeDtypeStruct((F, C * N), f32),
        ),
    )(x, x.T, wqp, bqp, WK_w, WK_b[:, None])

    pe_rs = position_embedding.reshape(C, N * N, GEO)

    bn = 64
    nb = N // bn
    out = pl.pallas_call(
        _attn_kernel,
        grid=(C, nb),
        in_specs=[
            pl.BlockSpec((1, bn * N, GEO), lambda c, i: (c, i, 0)),
            pl.BlockSpec((1, bn, N), lambda c, i: (c, i, 0)),
            pl.BlockSpec((bn, F), lambda c, i: (c * nb + i, 0)),
            pl.BlockSpec((N, F), lambda c, i: (c, 1)),
            pl.BlockSpec((F, N), lambda c, i: (0, c)),
            pl.BlockSpec((GEO, G), lambda c, i: (0, 0)),
            pl.BlockSpec((1, G), lambda c, i: (0, 0)),
            pl.BlockSpec((1, G * DG), lambda c, i: (0, 0)),
        ],
        out_specs=pl.BlockSpec((bn, G * DG), lambda c, i: (i, c)),
        out_shape=jax.ShapeDtypeStruct((N, C * G * DG), f32),
        scratch_shapes=[pltpu.VMEM((G, N, DG), f32)],
    )(pe_rs, iou, yqp, yqp, kt, WG_w.T, WG_b[None, :], conv_b[None, :])
    return out.reshape(N, C, G * DG)
